# baseline (speedup 1.0000x reference)
import functools

import jax
import jax.numpy as jnp
from jax import lax
from jax.experimental import pallas as pl
from jax.experimental.pallas import tpu as pltpu
from jax.experimental.pallas import tpu_sc as plsc

OUT_DIM = 1000
N = 16384

_NC = 2
_NS = 16
_NW = _NC * _NS
_ROWS_PER_W = N // _NW
_ZROWS = 64
_ZELEMS = _ZROWS * OUT_DIM
_NZDMA = _ROWS_PER_W // _ZROWS

_mesh = plsc.VectorSubcoreMesh(core_axis_name="c", subcore_axis_name="s")


@functools.partial(
    pl.kernel,
    mesh=_mesh,
    out_type=jax.ShapeDtypeStruct((N * OUT_DIM,), jnp.float32),
    scratch_types=[
        pltpu.VMEM((_ZELEMS,), jnp.float32),
        pltpu.VMEM((_ROWS_PER_W,), jnp.int32),
        pltpu.VMEM((_ROWS_PER_W,), jnp.int32),
        pltpu.VMEM((_ROWS_PER_W,), jnp.float32),
        pltpu.SemaphoreType.DMA,
        pltpu.SemaphoreType.DMA,
    ],
)
def _sc_onehot(idx_hbm, out_hbm, zbuf, idx_v, pos_v, ones_v, sem_z, sem_s):
    wid = lax.axis_index("s") * _NC + lax.axis_index("c")
    base = wid * _ROWS_PER_W

    pltpu.sync_copy(idx_hbm.at[pl.ds(base, _ROWS_PER_W)], idx_v)

    zeros16 = jnp.zeros((16,), jnp.float32)
    ones16 = jnp.ones((16,), jnp.float32)
    iota16 = lax.iota(jnp.int32, 16)

    def _zero_body(i, carry):
        b = i * 128
        for u in range(8):
            zbuf[pl.ds(b + u * 16, 16)] = zeros16
        return carry

    lax.fori_loop(0, _ZELEMS // 128, _zero_body, 0)

    copies = []
    for k in range(_NZDMA):
        dst = out_hbm.at[pl.ds((base + k * _ZROWS) * OUT_DIM, _ZELEMS)]
        copies.append(pltpu.async_copy(zbuf, dst, sem_z))

    def _pos_body(g, carry):
        off = g * 16
        row = base + off + iota16
        pos_v[pl.ds(off, 16)] = row * OUT_DIM + idx_v[pl.ds(off, 16)]
        ones_v[pl.ds(off, 16)] = ones16
        return carry

    lax.fori_loop(0, _ROWS_PER_W // 16, _pos_body, 0)

    for c in copies:
        c.wait()

    pltpu.async_copy(ones_v, out_hbm.at[pos_v], sem_s).wait()


def kernel(inputs):
    idx = inputs.astype(jnp.int32)
    flat = _sc_onehot(idx)
    return flat.reshape(N, OUT_DIM)
